# Initial kernel scaffold; baseline (speedup 1.0000x reference)
import functools

import jax
import jax.numpy as jnp
from jax import lax
from jax.experimental import pallas as pl
from jax.experimental.pallas import tpu as pltpu
from jax.experimental.pallas import tpu_sc as plsc

_NC, _NS = 2, 16
_NW = _NC * _NS


def _sc_gather(table, idx, chunk):
    (bf,) = idx.shape
    v, d = table.shape
    bpw = bf // _NW
    nch = bpw // chunk
    assert bpw * _NW == bf and nch * chunk == bpw

    mesh = plsc.VectorSubcoreMesh(core_axis_name="c", subcore_axis_name="s")

    @functools.partial(
        pl.kernel,
        out_type=jax.ShapeDtypeStruct((bf, d), table.dtype),
        mesh=mesh,
        scratch_types=[
            pltpu.VMEM((bpw,), jnp.int32),
            pltpu.VMEM((chunk, d), table.dtype),
            pltpu.SemaphoreType.DMA,
        ],
    )
    def k(table_hbm, idx_hbm, out_hbm, idx_v, buf, gsem):
        wid = lax.axis_index("s") * _NC + lax.axis_index("c")
        base = wid * bpw
        pltpu.sync_copy(idx_hbm.at[pl.ds(base, bpw)], idx_v)

        def body(j, carry):
            off = j * chunk
            pltpu.async_copy(
                table_hbm.at[idx_v.at[pl.ds(off, chunk)]], buf, gsem
            ).wait()
            pltpu.sync_copy(buf, out_hbm.at[pl.ds(base + off, chunk)])
            return carry

        lax.fori_loop(0, nch, body, 0)

    return k(table, idx)


def kernel(indices, embedding):
    b, f = indices.shape
    v, s, d = embedding.shape
    idx = indices.reshape(-1).astype(jnp.int32)
    table = embedding.reshape(v, s * d)
    out = _sc_gather(table, idx, 128)
    return out.reshape(b, f, s, d)


# speedup vs baseline: 1.4377x; 1.4377x over previous
import functools

import jax
import jax.numpy as jnp
from jax import lax
from jax.experimental import pallas as pl
from jax.experimental.pallas import tpu as pltpu
from jax.experimental.pallas import tpu_sc as plsc

_NC, _NS = 2, 16
_NW = _NC * _NS


def _sc_gather(table, idx, chunk):
    (bf,) = idx.shape
    v, d = table.shape
    bpw = bf // _NW
    nch = bpw // chunk
    assert bpw * _NW == bf and nch * chunk == bpw

    mesh = plsc.VectorSubcoreMesh(core_axis_name="c", subcore_axis_name="s")

    @functools.partial(
        pl.kernel,
        out_type=jax.ShapeDtypeStruct((bf, d), table.dtype),
        mesh=mesh,
        scratch_types=[
            pltpu.VMEM((bpw,), jnp.int32),
            pltpu.VMEM((chunk, d), table.dtype),
            pltpu.SemaphoreType.DMA,
        ],
        compiler_params=pltpu.CompilerParams(use_tc_tiling_on_sc=False),
    )
    def k(table_hbm, idx_hbm, out_hbm, idx_v, buf, gsem):
        wid = lax.axis_index("s") * _NC + lax.axis_index("c")
        base = wid * bpw
        pltpu.sync_copy(idx_hbm.at[pl.ds(base, bpw)], idx_v)

        def body(j, carry):
            off = j * chunk
            pltpu.async_copy(
                table_hbm.at[idx_v.at[pl.ds(off, chunk)]], buf, gsem
            ).wait()
            pltpu.sync_copy(buf, out_hbm.at[pl.ds(base + off, chunk)])
            return carry

        lax.fori_loop(0, nch, body, 0)

    return k(table, idx)


def kernel(indices, embedding):
    b, f = indices.shape
    v, s, d = embedding.shape
    idx = indices.reshape(-1).astype(jnp.int32)
    table = embedding.reshape(v, s * d)
    out = _sc_gather(table, idx, 128)
    return out.reshape(b, f, s, d)


# speedup vs baseline: 1.5401x; 1.0713x over previous
import functools

import jax
import jax.numpy as jnp
from jax import lax
from jax.experimental import pallas as pl
from jax.experimental.pallas import tpu as pltpu
from jax.experimental.pallas import tpu_sc as plsc

_NC, _NS = 2, 16
_NW = _NC * _NS


def _sc_gather(table, idx, chunk):
    (bf,) = idx.shape
    v, d = table.shape
    bpw = bf // _NW
    nch = bpw // chunk
    assert bpw * _NW == bf and nch * chunk == bpw

    mesh = plsc.VectorSubcoreMesh(core_axis_name="c", subcore_axis_name="s")

    @functools.partial(
        pl.kernel,
        out_type=jax.ShapeDtypeStruct((bf, d), table.dtype),
        mesh=mesh,
        scratch_types=[
            pltpu.VMEM((bpw,), jnp.int32),
            pltpu.VMEM((chunk, d), table.dtype),
            pltpu.SemaphoreType.DMA,
        ],
        compiler_params=pltpu.CompilerParams(use_tc_tiling_on_sc=False),
    )
    def k(table_hbm, idx_hbm, out_hbm, idx_v, buf, gsem):
        wid = lax.axis_index("s") * _NC + lax.axis_index("c")
        base = wid * bpw
        pltpu.sync_copy(idx_hbm.at[pl.ds(base, bpw)], idx_v)

        def body(j, carry):
            off = j * chunk
            pltpu.async_copy(
                table_hbm.at[idx_v.at[pl.ds(off, chunk)]], buf, gsem
            ).wait()
            pltpu.sync_copy(buf, out_hbm.at[pl.ds(base + off, chunk)])
            return carry

        lax.fori_loop(0, nch, body, 0)

    return k(table, idx)


def kernel(indices, embedding):
    b, f = indices.shape
    v, s, d = embedding.shape
    idx = indices.reshape(-1).astype(jnp.int32)
    table = embedding.reshape(v, s * d)
    out = _sc_gather(table, idx, 512)
    return out.reshape(b, f, s, d)


# speedup vs baseline: 1.5758x; 1.0232x over previous
import functools

import jax
import jax.numpy as jnp
from jax import lax
from jax.experimental import pallas as pl
from jax.experimental.pallas import tpu as pltpu
from jax.experimental.pallas import tpu_sc as plsc

_NC, _NS = 2, 16
_NW = _NC * _NS


def _sc_gather(table, idx, chunk, nbuf):
    (bf,) = idx.shape
    v, d = table.shape
    bpw = bf // _NW
    nch = bpw // chunk
    ngrp = nch // nbuf
    assert bpw * _NW == bf and nch * chunk == bpw and ngrp * nbuf == nch

    mesh = plsc.VectorSubcoreMesh(core_axis_name="c", subcore_axis_name="s")

    @functools.partial(
        pl.kernel,
        out_type=jax.ShapeDtypeStruct((bf, d), table.dtype),
        mesh=mesh,
        scratch_types=[
            pltpu.VMEM((bpw,), jnp.int32),
            [pltpu.VMEM((chunk, d), table.dtype) for _ in range(nbuf)],
            [pltpu.SemaphoreType.DMA for _ in range(nbuf)],
            [pltpu.SemaphoreType.DMA for _ in range(nbuf)],
        ],
        compiler_params=pltpu.CompilerParams(use_tc_tiling_on_sc=False),
    )
    def k(table_hbm, idx_hbm, out_hbm, idx_v, bufs, gsems, osems):
        wid = lax.axis_index("s") * _NC + lax.axis_index("c")
        base = wid * bpw
        pltpu.sync_copy(idx_hbm.at[pl.ds(base, bpw)], idx_v)

        def group(g, carry):
            goff = g * (nbuf * chunk)
            for b in range(nbuf):
                off = goff + b * chunk
                pltpu.async_copy(
                    table_hbm.at[idx_v.at[pl.ds(off, chunk)]], bufs[b], gsems[b]
                )
            for b in range(nbuf):
                off = goff + b * chunk
                pltpu.make_async_copy(
                    table_hbm.at[idx_v.at[pl.ds(off, chunk)]], bufs[b], gsems[b]
                ).wait()
                pltpu.async_copy(
                    bufs[b], out_hbm.at[pl.ds(base + off, chunk)], osems[b]
                )
            for b in range(nbuf):
                off = goff + b * chunk
                pltpu.make_async_copy(
                    bufs[b], out_hbm.at[pl.ds(base + off, chunk)], osems[b]
                ).wait()
            return carry

        lax.fori_loop(0, ngrp, group, 0)

    return k(table, idx)


def kernel(indices, embedding):
    b, f = indices.shape
    v, s, d = embedding.shape
    idx = indices.reshape(-1).astype(jnp.int32)
    table = embedding.reshape(v, s * d)
    out = _sc_gather(table, idx, 416, 4)
    return out.reshape(b, f, s, d)
